# baseline (speedup 1.0000x reference)
import functools

import jax
import jax.numpy as jnp
from jax import lax
from jax.experimental import pallas as pl
from jax.experimental.pallas import tpu as pltpu
from jax.experimental.pallas import tpu_sc as plsc

B, F, D = 4096, 100, 128
NC, NS = 2, 16
NW = NC * NS
RPW = B // NW
NSEM = 10
G = F // NSEM

_mesh = plsc.VectorSubcoreMesh(core_axis_name="c", subcore_axis_name="s")


@functools.partial(
    pl.kernel,
    mesh=_mesh,
    out_type=jax.ShapeDtypeStruct((F, B, D), jnp.float32),
    scratch_types=[pltpu.SemaphoreType.DMA for _ in range(NSEM)],
)
def _rev_copy(x_hbm, out_hbm, *sems):
    wid = lax.axis_index("s") * NC + lax.axis_index("c")
    r0 = wid * RPW

    def start(k, f):
        pltpu.async_copy(x_hbm.at[F - 1 - f].at[pl.ds(r0, RPW)],
                         out_hbm.at[f].at[pl.ds(r0, RPW)], sems[k])

    def wait(k):
        pltpu.make_async_copy(x_hbm.at[0].at[pl.ds(r0, RPW)],
                              out_hbm.at[0].at[pl.ds(r0, RPW)],
                              sems[k]).wait()

    for k in range(NSEM):
        start(k, k)

    def body(g, carry):
        for k in range(NSEM):
            wait(k)
            start(k, g * NSEM + k)
        return carry

    lax.fori_loop(1, G, body, 0)
    for k in range(NSEM):
        wait(k)


def kernel(x):
    out_t = _rev_copy(x.transpose(1, 0, 2))
    return out_t.transpose(1, 0, 2)


# speedup vs baseline: 38.7544x; 38.7544x over previous
import functools

import jax
import jax.numpy as jnp
from jax import lax
from jax.experimental import pallas as pl
from jax.experimental.pallas import tpu as pltpu
from jax.experimental.pallas import tpu_sc as plsc

B, F, D = 4096, 100, 128
NC, NS = 2, 16
NW = NC * NS
RPW = B // NW
NBUF = 4
G = F // NBUF

_mesh = plsc.VectorSubcoreMesh(core_axis_name="c", subcore_axis_name="s")


@functools.partial(
    pl.kernel,
    mesh=_mesh,
    out_type=jax.ShapeDtypeStruct((F, B, D), jnp.float32),
    scratch_types=(
        [pltpu.VMEM((RPW, D), jnp.float32) for _ in range(NBUF)]
        + [pltpu.SemaphoreType.DMA for _ in range(2 * NBUF)]
    ),
)
def _rev_copy(x_hbm, out_hbm, *refs):
    buf = refs[0:NBUF]
    rsem = refs[NBUF:2 * NBUF]
    wsem = refs[2 * NBUF:3 * NBUF]
    wid = lax.axis_index("s") * NC + lax.axis_index("c")
    r0 = wid * RPW

    def start_read(k, f):
        pltpu.async_copy(x_hbm.at[F - 1 - f].at[pl.ds(r0, RPW)],
                         buf[k], rsem[k])

    def wait_read(k):
        pltpu.make_async_copy(x_hbm.at[0].at[pl.ds(r0, RPW)],
                              buf[k], rsem[k]).wait()

    def start_write(k, f):
        pltpu.async_copy(buf[k], out_hbm.at[f].at[pl.ds(r0, RPW)], wsem[k])

    def wait_write(k):
        pltpu.make_async_copy(buf[k], out_hbm.at[0].at[pl.ds(r0, RPW)],
                              wsem[k]).wait()

    for k in range(NBUF):
        start_read(k, k)

    def body(g, carry):
        f0 = g * NBUF
        for k in range(NBUF):
            wait_read(k)
            start_write(k, f0 + k)

        @pl.when(g < G - 1)
        def _next():
            for k in range(NBUF):
                wait_write(k)
                start_read(k, f0 + NBUF + k)

        return carry

    lax.fori_loop(0, G, body, 0)
    for k in range(NBUF):
        wait_write(k)


def kernel(x):
    out_t = _rev_copy(x.transpose(1, 0, 2))
    return out_t.transpose(1, 0, 2)
